# baseline (speedup 1.0000x reference)
import jax
import jax.numpy as jnp
from jax.experimental import pallas as pl
from jax.experimental.pallas import tpu as pltpu

_NUM_EMBED = 1024
_EMBED_DIM = 64
_COMMIT = 0.25


def _vq_kernel(z_ref, w_ref, zq_ref, idx_ref, sse_ref, zs_ref):
    for h in range(32):
        zs_ref[:, h * 32:(h + 1) * 32] = z_ref[0, :, h, :]
    z = zs_ref[...]
    w = w_ref[...]
    wsq = jnp.sum(w * w, axis=1, keepdims=True)
    zsq = jnp.sum(z * z, axis=0, keepdims=True)
    mm = jax.lax.dot_general(-2.0 * w, z, (((1,), (0,)), ((), ())),
                             preferred_element_type=jnp.float32)
    scores = (zsq + wsq) + mm
    minv = jnp.min(scores, axis=0, keepdims=True)
    cio = jax.lax.broadcasted_iota(jnp.int32, scores.shape, 0)
    idx = jnp.min(jnp.where(scores == minv, cio, jnp.int32(2**30)), axis=0)
    idx_ref[0, 0, :] = idx
    onehot = (cio == idx[None, :]).astype(jnp.float32)
    zq = jax.lax.dot_general(w, onehot, (((0,), (0,)), ((), ())),
                             preferred_element_type=jnp.float32)
    st = z + (zq - z)
    for h in range(32):
        zq_ref[0, :, h, :] = st[:, h * 32:(h + 1) * 32]
    sse_ref[0] = jnp.full((8, 128), jnp.sum((zq - z) ** 2), jnp.float32)


def kernel(z_e, W):
    B, D, H, Wd = z_e.shape
    T = H * Wd
    zq4, idx3, sse = pl.pallas_call(
        _vq_kernel,
        grid=(B,),
        in_specs=[
            pl.BlockSpec((1, D, H, Wd), lambda b: (b, 0, 0, 0)),
            pl.BlockSpec((_NUM_EMBED, D), lambda b: (0, 0)),
        ],
        out_specs=[
            pl.BlockSpec((1, D, H, Wd), lambda b: (b, 0, 0, 0)),
            pl.BlockSpec((1, 1, T), lambda b: (b, 0, 0)),
            pl.BlockSpec((1, 8, 128), lambda b: (b, 0, 0)),
        ],
        out_shape=[
            jax.ShapeDtypeStruct((B, D, H, Wd), jnp.float32),
            jax.ShapeDtypeStruct((B, 1, T), jnp.int32),
            jax.ShapeDtypeStruct((B, 8, 128), jnp.float32),
        ],
        scratch_shapes=[pltpu.VMEM((_EMBED_DIM, T), jnp.float32)],
        compiler_params=pltpu.CompilerParams(
            dimension_semantics=("parallel",),
        ),
    )(z_e, W)
    indices = idx3.reshape(B, H, Wd)
    vq_loss = jnp.sum(sse[:, 0, 0]) / jnp.float32(B * D * T)
    commitment_loss = jnp.float32(_COMMIT) * vq_loss
    return (zq4, indices, vq_loss, commitment_loss)


# speedup vs baseline: 1.8814x; 1.8814x over previous
import jax
import jax.numpy as jnp
from jax.experimental import pallas as pl
from jax.experimental.pallas import tpu as pltpu

_NUM_EMBED = 1024
_EMBED_DIM = 64
_COMMIT = 0.25


def _vq_kernel(z_ref, w_ref, zq_ref, idx_ref, sse_ref):
    z = z_ref[0]
    w = w_ref[...]
    wsq = jnp.sum(w * w, axis=1, keepdims=True)
    zsq = jnp.sum(z * z, axis=0, keepdims=True)
    mm = jax.lax.dot_general(-2.0 * w, z, (((1,), (0,)), ((), ())),
                             preferred_element_type=jnp.float32)
    scores = (zsq + wsq) + mm
    minv = jnp.min(scores, axis=0, keepdims=True)
    cio = jax.lax.broadcasted_iota(jnp.int32, scores.shape, 0)
    idx = jnp.min(jnp.where(scores == minv, cio, jnp.int32(2**30)), axis=0)
    idx_ref[0, 0, :] = idx
    onehot = (cio == idx[None, :]).astype(jnp.float32)
    zq = jax.lax.dot_general(w, onehot, (((0,), (0,)), ((), ())),
                             preferred_element_type=jnp.float32)
    zq_ref[0] = zq
    sse_ref[0] = jnp.full((8, 128), jnp.sum((zq - z) ** 2), jnp.float32)


def kernel(z_e, W):
    B, D, H, Wd = z_e.shape
    T = H * Wd
    z3 = z_e.reshape(B, D, T)
    zq3, idx3, sse = pl.pallas_call(
        _vq_kernel,
        grid=(B,),
        in_specs=[
            pl.BlockSpec((1, D, T), lambda b: (b, 0, 0)),
            pl.BlockSpec((_NUM_EMBED, D), lambda b: (0, 0)),
        ],
        out_specs=[
            pl.BlockSpec((1, D, T), lambda b: (b, 0, 0)),
            pl.BlockSpec((1, 1, T), lambda b: (b, 0, 0)),
            pl.BlockSpec((1, 8, 128), lambda b: (b, 0, 0)),
        ],
        out_shape=[
            jax.ShapeDtypeStruct((B, D, T), jnp.float32),
            jax.ShapeDtypeStruct((B, 1, T), jnp.int32),
            jax.ShapeDtypeStruct((B, 8, 128), jnp.float32),
        ],
        compiler_params=pltpu.CompilerParams(
            dimension_semantics=("parallel",),
        ),
    )(z3, W)
    z_q_st = zq3.reshape(B, D, H, Wd)
    indices = idx3.reshape(B, H, Wd)
    vq_loss = jnp.sum(sse[:, 0, 0]) / jnp.float32(B * D * T)
    commitment_loss = jnp.float32(_COMMIT) * vq_loss
    return (z_q_st, indices, vq_loss, commitment_loss)
